# baseline (speedup 1.0000x reference)
import jax
import jax.numpy as jnp
from jax import lax
from jax.experimental import pallas as pl
from jax.experimental.pallas import tpu as pltpu

_TILE_ROWS = 43008


def _l2_body(x_ref, o_ref):
    x = x_ref[...]
    ssq = jnp.sum(x * x, axis=-1, keepdims=True)
    o_ref[...] = x * lax.rsqrt(ssq)


def kernel(x):
    shape = x.shape
    d = shape[-1]
    xf = x.reshape(-1, d)
    m = xf.shape[0]
    tm = min(_TILE_ROWS, m)
    out = pl.pallas_call(
        _l2_body,
        out_shape=jax.ShapeDtypeStruct((m, d), x.dtype),
        grid=(pl.cdiv(m, tm),),
        in_specs=[pl.BlockSpec((tm, d), lambda i: (i, 0))],
        out_specs=pl.BlockSpec((tm, d), lambda i: (i, 0),
                               pipeline_mode=pl.Buffered(buffer_count=1)),
        compiler_params=pltpu.CompilerParams(
            dimension_semantics=("parallel",),
            vmem_limit_bytes=64 * 1024 * 1024,
        ),
    )(xf)
    return out.reshape(shape)


# speedup vs baseline: 1.1662x; 1.1662x over previous
import jax
import jax.numpy as jnp
from jax import lax
from jax.experimental import pallas as pl
from jax.experimental.pallas import tpu as pltpu

_TILE_ROWS = 32256


def _l2_body(x_ref, o_ref):
    x = x_ref[...]
    ssq = jnp.sum(x * x, axis=-1, keepdims=True)
    o_ref[...] = x * lax.rsqrt(ssq)


def kernel(x):
    shape = x.shape
    d = shape[-1]
    xf = x.reshape(-1, d)
    m = xf.shape[0]
    tm = min(_TILE_ROWS, m)
    out = pl.pallas_call(
        _l2_body,
        out_shape=jax.ShapeDtypeStruct((m, d), x.dtype),
        grid=(pl.cdiv(m, tm),),
        in_specs=[pl.BlockSpec((tm, d), lambda i: (i, 0))],
        out_specs=pl.BlockSpec((tm, d), lambda i: (i, 0)),
        compiler_params=pltpu.CompilerParams(
            dimension_semantics=("parallel",),
            vmem_limit_bytes=64 * 1024 * 1024,
        ),
    )(xf)
    return out.reshape(shape)
